# Initial kernel scaffold; baseline (speedup 1.0000x reference)
import jax
import jax.numpy as jnp
from jax.experimental import pallas as pl


def kernel(box, cls, sizes):
    raise NotImplementedError("write your pallas kernel here")


# speedup vs baseline: 1.3247x; 1.3247x over previous
import jax
import jax.numpy as jnp
from jax.experimental import pallas as pl
from jax.experimental.pallas import tpu as pltpu

B = 8
N = 20000
C = 80
NMS_THRESHOLD = 0.5
SCORE_THRESHOLD = 0.05
MAX_DETECTIONS = 300
M = 1000

_NT = 2000


def _prep_kernel(box_ref, cls_ref, sizes_ref, clip_ref, score_ref, label_ref):
    bx = box_ref[0]
    cl = cls_ref[0]
    s2 = sizes_ref[pl.ds(pl.program_id(0), 1), :]

    li2 = jax.lax.broadcasted_iota(jnp.int32, (1, 2), 1)
    w = jnp.max(jnp.where(li2 == 1, s2, 0.0))
    h = jnp.max(jnp.where(li2 == 0, s2, 0.0))
    li4 = jax.lax.broadcasted_iota(jnp.int32, (1, 4), 1)
    bound = jnp.where(li4 % 2 == 0, w, h)
    clip_ref[0] = jnp.clip(bx, 0.0, bound)

    scores = jnp.max(cl, axis=1, keepdims=True)
    lane = jax.lax.broadcasted_iota(jnp.int32, cl.shape, 1)
    labels = jnp.min(jnp.where(cl == scores, lane, C), axis=1, keepdims=True)
    score_ref[0] = jnp.where(scores > SCORE_THRESHOLD, scores, -1e9)
    label_ref[0] = labels


def _nms_kernel(bcol_ref, brow_ref, s_ref, keep_ref, iou_ref):
    bc = bcol_ref[0]
    br = brow_ref[0]
    x1c, y1c = bc[:, 0:1], bc[:, 1:2]
    x2c, y2c = bc[:, 2:3], bc[:, 3:4]
    x1r, y1r = br[0:1, :], br[1:2, :]
    x2r, y2r = br[2:3, :], br[3:4, :]
    area_c = jnp.maximum(x2c - x1c, 0.0) * jnp.maximum(y2c - y1c, 0.0)
    area_r = jnp.maximum(x2r - x1r, 0.0) * jnp.maximum(y2r - y1r, 0.0)
    xx1 = jnp.maximum(x1c, x1r)
    yy1 = jnp.maximum(y1c, y1r)
    xx2 = jnp.minimum(x2c, x2r)
    yy2 = jnp.minimum(y2c, y2r)
    inter = jnp.maximum(xx2 - xx1, 0.0) * jnp.maximum(yy2 - yy1, 0.0)
    union = area_c + area_r - inter
    iou_ref[...] = inter / jnp.maximum(union, 1e-9)

    idx = jax.lax.broadcasted_iota(jnp.int32, (1, M), 1)
    keep_ref[0] = jnp.where(s_ref[0] > SCORE_THRESHOLD, 1.0, 0.0)

    def body(i, carry):
        keepf = keep_ref[0]
        row = iou_ref[pl.ds(i, 1), :]
        ki = jnp.max(jnp.where(idx == i, keepf, 0.0))
        sup = (row > NMS_THRESHOLD) & (idx > i) & (ki > 0.5)
        keep_ref[0] = jnp.where(sup, 0.0, keepf)
        return carry

    jax.lax.fori_loop(0, M, body, 0)


def kernel(box, cls, sizes):
    clipped, score, label = pl.pallas_call(
        _prep_kernel,
        grid=(B, N // _NT),
        in_specs=[
            pl.BlockSpec((1, _NT, 4), lambda b, n: (b, n, 0)),
            pl.BlockSpec((1, _NT, C), lambda b, n: (b, n, 0)),
            pl.BlockSpec((B, 2), lambda b, n: (0, 0)),
        ],
        out_specs=[
            pl.BlockSpec((1, _NT, 4), lambda b, n: (b, n, 0)),
            pl.BlockSpec((1, _NT, 1), lambda b, n: (b, n, 0)),
            pl.BlockSpec((1, _NT, 1), lambda b, n: (b, n, 0)),
        ],
        out_shape=[
            jax.ShapeDtypeStruct((B, N, 4), jnp.float32),
            jax.ShapeDtypeStruct((B, N, 1), jnp.float32),
            jax.ShapeDtypeStruct((B, N, 1), jnp.int32),
        ],
    )(box, cls, sizes)

    masked = score.reshape(B, N)
    labels = label.reshape(B, N)
    top_s, top_i = jax.lax.top_k(masked, M)
    b_g = jnp.take_along_axis(clipped, top_i[:, :, None], axis=1)
    lab_g = jnp.take_along_axis(labels, top_i, axis=1)
    b_row = jnp.transpose(b_g, (0, 2, 1))

    keep = pl.pallas_call(
        _nms_kernel,
        grid=(B,),
        in_specs=[
            pl.BlockSpec((1, M, 4), lambda b: (b, 0, 0)),
            pl.BlockSpec((1, 4, M), lambda b: (b, 0, 0)),
            pl.BlockSpec((1, 1, M), lambda b: (b, 0, 0)),
        ],
        out_specs=pl.BlockSpec((1, 1, M), lambda b: (b, 0, 0)),
        out_shape=jax.ShapeDtypeStruct((B, 1, M), jnp.float32),
        scratch_shapes=[pltpu.VMEM((M, M), jnp.float32)],
    )(b_g, b_row, top_s.reshape(B, 1, M))
    keep = keep.reshape(B, M)

    final = jnp.where(keep > 0.5, top_s, -1e9)
    det_s, det_i = jax.lax.top_k(final, MAX_DETECTIONS)
    det_valid = det_s > SCORE_THRESHOLD
    out_b = jnp.where(det_valid[:, :, None],
                      jnp.take_along_axis(b_g, det_i[:, :, None], axis=1), -1.0)
    out_s = jnp.where(det_valid, det_s, -1.0)[:, :, None]
    out_l = jnp.where(det_valid,
                      jnp.take_along_axis(lab_g, det_i, axis=1).astype(jnp.float32),
                      -1.0)[:, :, None]
    return (out_b, out_s, out_l)


# speedup vs baseline: 1.3248x; 1.0000x over previous
import jax
import jax.numpy as jnp
from jax.experimental import pallas as pl
from jax.experimental.pallas import tpu as pltpu

B = 8
N = 20000
C = 80
NMS_THRESHOLD = 0.5
SCORE_THRESHOLD = 0.05
MAX_DETECTIONS = 300
M = 1000

_NT = 2000


def _prep_kernel(box_ref, cls_ref, sizes_ref, clip_ref, score_ref, label_ref):
    bx = box_ref[0]
    cl = cls_ref[0]
    s2 = sizes_ref[pl.ds(pl.program_id(0), 1), :]

    li2 = jax.lax.broadcasted_iota(jnp.int32, (1, 2), 1)
    w = jnp.max(jnp.where(li2 == 1, s2, 0.0))
    h = jnp.max(jnp.where(li2 == 0, s2, 0.0))
    li4 = jax.lax.broadcasted_iota(jnp.int32, (1, 4), 1)
    bound = jnp.where(li4 % 2 == 0, w, h)
    clip_ref[0] = jnp.clip(bx, 0.0, bound)

    scores = jnp.max(cl, axis=1, keepdims=True)
    lane = jax.lax.broadcasted_iota(jnp.int32, cl.shape, 1)
    labels = jnp.min(jnp.where(cl == scores, lane, C), axis=1, keepdims=True)
    score_ref[0] = jnp.where(scores > SCORE_THRESHOLD, scores, -1e9)
    label_ref[0] = labels


def _nms_kernel(bcol_ref, brow_ref, s_ref, keep_ref, iou_ref):
    bc = bcol_ref[0]
    br = brow_ref[0]
    x1c, y1c = bc[:, 0:1], bc[:, 1:2]
    x2c, y2c = bc[:, 2:3], bc[:, 3:4]
    x1r, y1r = br[0:1, :], br[1:2, :]
    x2r, y2r = br[2:3, :], br[3:4, :]
    area_c = jnp.maximum(x2c - x1c, 0.0) * jnp.maximum(y2c - y1c, 0.0)
    area_r = jnp.maximum(x2r - x1r, 0.0) * jnp.maximum(y2r - y1r, 0.0)
    xx1 = jnp.maximum(x1c, x1r)
    yy1 = jnp.maximum(y1c, y1r)
    xx2 = jnp.minimum(x2c, x2r)
    yy2 = jnp.minimum(y2c, y2r)
    inter = jnp.maximum(xx2 - xx1, 0.0) * jnp.maximum(yy2 - yy1, 0.0)
    union = area_c + area_r - inter
    iou_ref[...] = inter / jnp.maximum(union, 1e-9)

    idx = jax.lax.broadcasted_iota(jnp.int32, (1, M), 1)
    keep_ref[0] = jnp.where(s_ref[0] > SCORE_THRESHOLD, 1.0, 0.0)

    def body(i, carry):
        keepf = keep_ref[0]
        row = iou_ref[pl.ds(i, 1), :]
        ki = jnp.max(jnp.where(idx == i, keepf, 0.0))
        sup = (row > NMS_THRESHOLD) & (idx > i) & (ki > 0.5)
        keep_ref[0] = jnp.where(sup, 0.0, keepf)
        return carry

    jax.lax.fori_loop(0, M, body, 0)


def kernel(box, cls, sizes):
    clipped, score, label = pl.pallas_call(
        _prep_kernel,
        grid=(B, N // _NT),
        in_specs=[
            pl.BlockSpec((1, _NT, 4), lambda b, n: (b, n, 0)),
            pl.BlockSpec((1, _NT, C), lambda b, n: (b, n, 0)),
            pl.BlockSpec((B, 2), lambda b, n: (0, 0)),
        ],
        out_specs=[
            pl.BlockSpec((1, _NT, 4), lambda b, n: (b, n, 0)),
            pl.BlockSpec((1, _NT, 1), lambda b, n: (b, n, 0)),
            pl.BlockSpec((1, _NT, 1), lambda b, n: (b, n, 0)),
        ],
        out_shape=[
            jax.ShapeDtypeStruct((B, N, 4), jnp.float32),
            jax.ShapeDtypeStruct((B, N, 1), jnp.float32),
            jax.ShapeDtypeStruct((B, N, 1), jnp.int32),
        ],
        compiler_params=pltpu.CompilerParams(
            dimension_semantics=("parallel", "arbitrary")),
    )(box, cls, sizes)

    masked = score.reshape(B, N)
    labels = label.reshape(B, N)
    top_s, top_i = jax.lax.top_k(masked, M)
    b_g = jnp.take_along_axis(clipped, top_i[:, :, None], axis=1)
    lab_g = jnp.take_along_axis(labels, top_i, axis=1)
    b_row = jnp.transpose(b_g, (0, 2, 1))

    keep = pl.pallas_call(
        _nms_kernel,
        grid=(B,),
        in_specs=[
            pl.BlockSpec((1, M, 4), lambda b: (b, 0, 0)),
            pl.BlockSpec((1, 4, M), lambda b: (b, 0, 0)),
            pl.BlockSpec((1, 1, M), lambda b: (b, 0, 0)),
        ],
        out_specs=pl.BlockSpec((1, 1, M), lambda b: (b, 0, 0)),
        out_shape=jax.ShapeDtypeStruct((B, 1, M), jnp.float32),
        scratch_shapes=[pltpu.VMEM((M, M), jnp.float32)],
        compiler_params=pltpu.CompilerParams(
            dimension_semantics=("parallel",)),
    )(b_g, b_row, top_s.reshape(B, 1, M))
    keep = keep.reshape(B, M)

    final = jnp.where(keep > 0.5, top_s, -1e9)
    det_s, det_i = jax.lax.top_k(final, MAX_DETECTIONS)
    det_valid = det_s > SCORE_THRESHOLD
    out_b = jnp.where(det_valid[:, :, None],
                      jnp.take_along_axis(b_g, det_i[:, :, None], axis=1), -1.0)
    out_s = jnp.where(det_valid, det_s, -1.0)[:, :, None]
    out_l = jnp.where(det_valid,
                      jnp.take_along_axis(lab_g, det_i, axis=1).astype(jnp.float32),
                      -1.0)[:, :, None]
    return (out_b, out_s, out_l)
